# baseline (speedup 1.0000x reference)
import functools

import jax
import jax.numpy as jnp
from jax import lax
from jax.experimental import pallas as pl
from jax.experimental.pallas import tpu as pltpu
from jax.experimental.pallas import tpu_sc as plsc

B = 4096
HIST = 200
D = 64
NC = 2
NS = 16
NW = NC * NS
BPW = B // NW
CHUNK = 40
NCHUNK = HIST // CHUNK
NLANE = 16
NVREG = D // NLANE


def _sc_body(idx_hbm, table_hbm, out_hbm, idx_v, *rest):
  rows_flat = rest[:2 * NCHUNK]
  out_v = rest[2 * NCHUNK]
  sems = rest[2 * NCHUNK + 1:]
  rows = (rows_flat[:NCHUNK], rows_flat[NCHUNK:])

  cid = lax.axis_index("c")
  sid = lax.axis_index("s")
  wid = sid * NC + cid
  base = wid * BPW

  pltpu.sync_copy(idx_hbm.at[pl.ds(base * HIST, BPW * HIST)], idx_v)

  def start_row(i, s):
    for c in range(NCHUNK):
      pltpu.async_copy(
          table_hbm.at[idx_v.at[pl.ds(i * HIST + c * CHUNK, CHUNK)]],
          rows[s][c], sems[s])

  def wait_row(s):
    for c in range(NCHUNK):
      pltpu.make_async_copy(
          table_hbm.at[pl.ds(0, CHUNK)], rows[s][c], sems[s]).wait()

  def accum_store(i, s):
    def inner(j, acc):
      for c in range(NCHUNK):
        acc = tuple(
            acc[k] + rows[s][c][j, pl.ds(k * NLANE, NLANE)]
            for k in range(NVREG))
      return acc
    zeros = tuple(jnp.zeros((NLANE,), jnp.float32) for _ in range(NVREG))
    acc = lax.fori_loop(0, CHUNK, inner, zeros)
    scale = jnp.float32(1.0 / HIST)
    for k in range(NVREG):
      out_v[i, pl.ds(k * NLANE, NLANE)] = acc[k] * scale

  start_row(0, 0)

  def body(p, _):
    i0 = 2 * p
    start_row(i0 + 1, 1)
    wait_row(0)
    accum_store(i0, 0)

    @pl.when(i0 + 2 < BPW)
    def _():
      start_row(i0 + 2, 0)

    wait_row(1)
    accum_store(i0 + 1, 1)
    return 0

  lax.fori_loop(0, BPW // 2, body, 0)
  pltpu.sync_copy(out_v, out_hbm.at[pl.ds(base, BPW)])


@functools.partial(jax.jit, static_argnums=())
def _sc_gather_mean(idx_flat, table):
  mesh = plsc.VectorSubcoreMesh(core_axis_name="c", subcore_axis_name="s")
  return pl.kernel(
      _sc_body,
      out_type=jax.ShapeDtypeStruct((B, D), jnp.float32),
      mesh=mesh,
      compiler_params=pltpu.CompilerParams(use_tc_tiling_on_sc=False),
      scratch_types=(
          [pltpu.VMEM((BPW * HIST,), jnp.int32)]
          + [pltpu.VMEM((CHUNK, D), jnp.float32) for _ in range(2 * NCHUNK)]
          + [pltpu.VMEM((BPW, D), jnp.float32)]
          + [pltpu.SemaphoreType.DMA, pltpu.SemaphoreType.DMA]
      ),
  )(idx_flat, table)


def _mlp_body(x_ref, w1_ref, b1_ref, w2_ref, b2_ref, o_ref):
  x = x_ref[...]
  h = jnp.dot(x, w1_ref[...], preferred_element_type=jnp.float32)
  h = jnp.maximum(h + b1_ref[...], 0.0)
  o_ref[...] = jnp.dot(h, w2_ref[...],
                       preferred_element_type=jnp.float32) + b2_ref[...]


def _mlp(avg, W1, b1, W2, b2):
  return pl.pallas_call(
      _mlp_body,
      out_shape=jax.ShapeDtypeStruct((B, b2.shape[-1]), jnp.float32),
  )(avg, W1, b1, W2, b2)


def kernel(word_indices, table, W1, b1, W2, b2):
  idx_flat = word_indices.reshape(-1).astype(jnp.int32)
  avg = _sc_gather_mean(idx_flat, table)
  return _mlp(avg, W1, b1.reshape(1, -1), W2, b2.reshape(1, -1))
